# baseline (speedup 1.0000x reference)
import functools

import jax
import jax.numpy as jnp
import numpy as np
from jax import lax
from jax.experimental import pallas as pl
from jax.experimental.pallas import tpu as pltpu
from jax.experimental.pallas import tpu_sc as plsc

VOCAB = 100000
HIDDEN = 128
LABELS = 100
BATCH = 4096
SEQ = 200

NC = 2
NS = 16
NW = NC * NS
ROWS_PER_W = BATCH // NW
HALF = SEQ // 2
WORDS = HIDDEN // 2
NLOAD = WORDS // 16
_MASK = np.int32(np.uint32(0xFFFF0000).view(np.int32))

VB = 10000


def _pack_table_tc(table):
    def pack_kernel(xa_ref, xb_ref, o_ref):
        def words(x):
            u = lax.bitcast_convert_type(x, jnp.uint32)
            return (u[:, :WORDS] >> 16) | (u[:, WORDS:] & jnp.uint32(0xFFFF0000))

        o_ref[...] = lax.bitcast_convert_type(
            jnp.concatenate([words(xa_ref[...]), words(xb_ref[...])], axis=1),
            jnp.int32)

    nblk = VOCAB // (2 * VB)
    return pl.pallas_call(
        pack_kernel,
        grid=(nblk,),
        in_specs=[
            pl.BlockSpec((VB, HIDDEN), lambda i: (i, 0)),
            pl.BlockSpec((VB, HIDDEN), lambda i: (i + nblk, 0)),
        ],
        out_specs=pl.BlockSpec((VB, HIDDEN), lambda i: (i, 0)),
        out_shape=jax.ShapeDtypeStruct((VOCAB // 2, HIDDEN), jnp.int32),
    )(table, table)


def _pooled_sum_sc(tab_i32, idx4):
    mesh = plsc.VectorSubcoreMesh(
        core_axis_name="c", subcore_axis_name="s", num_cores=NC, num_subcores=NS
    )

    @functools.partial(
        pl.kernel,
        out_type=jax.ShapeDtypeStruct((BATCH, HIDDEN), jnp.float32),
        mesh=mesh,
        scratch_types=[
            pltpu.VMEM((ROWS_PER_W, 2, HALF), jnp.int32),
            pltpu.VMEM((ROWS_PER_W, 2, HALF), jnp.int32),
            pltpu.VMEM((2, 2, HALF, WORDS), jnp.int32),
            pltpu.VMEM((ROWS_PER_W, HIDDEN), jnp.float32),
            pltpu.SemaphoreType.DMA,
            pltpu.SemaphoreType.DMA,
        ],
        compiler_params=pltpu.CompilerParams(use_tc_tiling_on_sc=False),
    )
    def k(table_hbm, idx_hbm, out_hbm, raw_v, idx_v, rows_v, out_v, semA, semB):
        wid = lax.axis_index("s") * NC + lax.axis_index("c")
        base = wid * ROWS_PER_W

        pltpu.sync_copy(idx_hbm.at[wid], raw_v)

        def remap_row(i):
            for h in range(2):
                for o in (0, 16, 32, 48, 64, 80, 84):
                    v = raw_v[i, h, pl.ds(o, 16)]
                    two = v + v
                    idx_v[i, h, pl.ds(o, 16)] = jnp.where(
                        v < VOCAB // 2, two, two - (VOCAB - 1))

        def start(i, p, sem):
            for h in range(2):
                pltpu.async_copy(
                    table_hbm.at[idx_v.at[i, h]], rows_v.at[p, h], sem)

        def wait(i, p, sem):
            for h in range(2):
                pltpu.make_async_copy(
                    table_hbm.at[idx_v.at[i, h]], rows_v.at[p, h], sem).wait()

        def acc_row(i, p):
            def acc_body(j, acc):
                out = list(acc)
                for h in range(2):
                    for l in range(NLOAD):
                        x = rows_v[p, h, j, pl.ds(16 * l, 16)]
                        lo = lax.bitcast_convert_type(x << 16, jnp.float32)
                        hi = lax.bitcast_convert_type(x & _MASK, jnp.float32)
                        out[l] = out[l] + lo
                        out[NLOAD + l] = out[NLOAD + l] + hi
                return tuple(out)

            acc0 = tuple(jnp.zeros((16,), jnp.float32) for _ in range(8))
            acc = lax.fori_loop(0, HALF, acc_body, acc0)
            for c in range(8):
                out_v[i, pl.ds(16 * c, 16)] = acc[c]

        remap_row(0)
        remap_row(1)
        start(0, 0, semA)

        def pair_body(kk, carry):
            i0 = 2 * kk

            @pl.when(i0 + 2 < ROWS_PER_W)
            def _():
                remap_row(i0 + 2)
                remap_row(i0 + 3)

            start(i0 + 1, 1, semB)
            wait(i0, 0, semA)
            acc_row(i0, 0)

            @pl.when(i0 + 2 < ROWS_PER_W)
            def _():
                start(i0 + 2, 0, semA)

            wait(i0 + 1, 1, semB)
            acc_row(i0 + 1, 1)
            return carry

        lax.fori_loop(0, ROWS_PER_W // 2, pair_body, 0)
        pltpu.sync_copy(out_v, out_hbm.at[pl.ds(base, ROWS_PER_W)])

    return k(tab_i32, idx4)


def _mlp_tc(pooled, inp, t0p, W1t, b1, W2t, b2, W3t, b3):
    BB = 2048

    def mlp_kernel(p_ref, idx_ref, t0_ref, w1_ref, b1_ref, w2_ref, b2_ref,
                   w3_ref, b3_ref, o_ref):
        cnt = jnp.sum((idx_ref[...] == 0).astype(jnp.float32), axis=1,
                      keepdims=True)
        bow = (p_ref[...] - cnt * t0_ref[...]) * (1.0 / SEQ)
        h = jnp.maximum(
            jnp.dot(bow, w1_ref[...], preferred_element_type=jnp.float32)
            + b1_ref[...], 0.0)
        h = jnp.maximum(
            jnp.dot(h, w2_ref[...], preferred_element_type=jnp.float32)
            + b2_ref[...], 0.0)
        o_ref[...] = (
            jnp.dot(h, w3_ref[...], preferred_element_type=jnp.float32)
            + b3_ref[...])

    return pl.pallas_call(
        mlp_kernel,
        grid=(BATCH // BB,),
        in_specs=[
            pl.BlockSpec((BB, HIDDEN), lambda i: (i, 0)),
            pl.BlockSpec((BB, SEQ), lambda i: (i, 0)),
            pl.BlockSpec((1, HIDDEN), lambda i: (0, 0)),
            pl.BlockSpec((HIDDEN, HIDDEN), lambda i: (0, 0)),
            pl.BlockSpec((1, HIDDEN), lambda i: (0, 0)),
            pl.BlockSpec((HIDDEN, HIDDEN), lambda i: (0, 0)),
            pl.BlockSpec((1, HIDDEN), lambda i: (0, 0)),
            pl.BlockSpec((HIDDEN, LABELS), lambda i: (0, 0)),
            pl.BlockSpec((1, LABELS), lambda i: (0, 0)),
        ],
        out_specs=pl.BlockSpec((BB, LABELS), lambda i: (i, 0)),
        out_shape=jax.ShapeDtypeStruct((BATCH, LABELS), jnp.float32),
    )(pooled, inp, t0p, W1t, b1, W2t, b2, W3t, b3)


def kernel(input, table, W1, b1, W2, b2, W3, b3):
    inp = input.astype(jnp.int32)
    idx4 = inp.reshape(NW, ROWS_PER_W, 2, HALF)
    tab_i32 = _pack_table_tc(table).reshape(VOCAB, WORDS)
    pooled = _pooled_sum_sc(tab_i32, idx4)
    w0 = tab_i32[0:1]
    t0p = jnp.concatenate([
        lax.bitcast_convert_type(w0 << 16, jnp.float32),
        lax.bitcast_convert_type(w0 & _MASK, jnp.float32),
    ], axis=1)
    return _mlp_tc(
        pooled, inp, t0p,
        W1.T, b1.reshape(1, HIDDEN),
        W2.T, b2.reshape(1, HIDDEN),
        W3.T, b3.reshape(1, LABELS),
    )


# speedup vs baseline: 1.0047x; 1.0047x over previous
import functools

import jax
import jax.numpy as jnp
import numpy as np
from jax import lax
from jax.experimental import pallas as pl
from jax.experimental.pallas import tpu as pltpu
from jax.experimental.pallas import tpu_sc as plsc

VOCAB = 100000
HIDDEN = 128
LABELS = 100
BATCH = 4096
SEQ = 200

NC = 2
NS = 16
NW = NC * NS
ROWS_PER_W = BATCH // NW
HALF = SEQ // 2
WORDS = HIDDEN // 2
NLOAD = WORDS // 16
_MASK = np.int32(np.uint32(0xFFFF0000).view(np.int32))

VB = 5000


def _pack_table_tc(table):
    def pack_kernel(xa_ref, xb_ref, o_ref):
        def words(x):
            u = lax.bitcast_convert_type(x, jnp.uint32)
            return (u[:, :WORDS] >> 16) | (u[:, WORDS:] & jnp.uint32(0xFFFF0000))

        o_ref[...] = lax.bitcast_convert_type(
            jnp.concatenate([words(xa_ref[...]), words(xb_ref[...])], axis=1),
            jnp.int32)

    nblk = VOCAB // (2 * VB)
    return pl.pallas_call(
        pack_kernel,
        grid=(nblk,),
        in_specs=[
            pl.BlockSpec((VB, HIDDEN), lambda i: (i, 0)),
            pl.BlockSpec((VB, HIDDEN), lambda i: (i + nblk, 0)),
        ],
        out_specs=pl.BlockSpec((VB, HIDDEN), lambda i: (i, 0)),
        out_shape=jax.ShapeDtypeStruct((VOCAB // 2, HIDDEN), jnp.int32),
    )(table, table)


def _pooled_sum_sc(tab_i32, idx4):
    mesh = plsc.VectorSubcoreMesh(
        core_axis_name="c", subcore_axis_name="s", num_cores=NC, num_subcores=NS
    )

    @functools.partial(
        pl.kernel,
        out_type=jax.ShapeDtypeStruct((BATCH, HIDDEN), jnp.float32),
        mesh=mesh,
        scratch_types=[
            pltpu.VMEM((ROWS_PER_W, 2, HALF), jnp.int32),
            pltpu.VMEM((ROWS_PER_W, 2, HALF), jnp.int32),
            pltpu.VMEM((2, 2, HALF, WORDS), jnp.int32),
            pltpu.VMEM((ROWS_PER_W, HIDDEN), jnp.float32),
            pltpu.SemaphoreType.DMA,
            pltpu.SemaphoreType.DMA,
        ],
        compiler_params=pltpu.CompilerParams(use_tc_tiling_on_sc=False),
    )
    def k(table_hbm, idx_hbm, out_hbm, raw_v, idx_v, rows_v, out_v, semA, semB):
        wid = lax.axis_index("s") * NC + lax.axis_index("c")
        base = wid * ROWS_PER_W

        pltpu.sync_copy(idx_hbm.at[wid], raw_v)

        def remap_row(i):
            for h in range(2):
                for o in (0, 16, 32, 48, 64, 80, 84):
                    v = raw_v[i, h, pl.ds(o, 16)]
                    two = v + v
                    idx_v[i, h, pl.ds(o, 16)] = jnp.where(
                        v < VOCAB // 2, two, two - (VOCAB - 1))

        def start(i, p, sem):
            for h in range(2):
                pltpu.async_copy(
                    table_hbm.at[idx_v.at[i, h]], rows_v.at[p, h], sem)

        def wait(i, p, sem):
            for h in range(2):
                pltpu.make_async_copy(
                    table_hbm.at[idx_v.at[i, h]], rows_v.at[p, h], sem).wait()

        def acc_row(i, p):
            def acc_body(j, acc):
                out = list(acc)
                for h in range(2):
                    for l in range(NLOAD):
                        x = rows_v[p, h, j, pl.ds(16 * l, 16)]
                        lo = lax.bitcast_convert_type(x << 16, jnp.float32)
                        hi = lax.bitcast_convert_type(x & _MASK, jnp.float32)
                        out[l] = out[l] + lo
                        out[NLOAD + l] = out[NLOAD + l] + hi
                return tuple(out)

            acc0 = tuple(jnp.zeros((16,), jnp.float32) for _ in range(8))
            acc = lax.fori_loop(0, HALF, acc_body, acc0)
            for c in range(8):
                out_v[i, pl.ds(16 * c, 16)] = acc[c]

        remap_row(0)
        remap_row(1)
        start(0, 0, semA)

        def pair_body(kk, carry):
            i0 = 2 * kk

            @pl.when(i0 + 2 < ROWS_PER_W)
            def _():
                remap_row(i0 + 2)
                remap_row(i0 + 3)

            start(i0 + 1, 1, semB)
            wait(i0, 0, semA)
            acc_row(i0, 0)

            @pl.when(i0 + 2 < ROWS_PER_W)
            def _():
                start(i0 + 2, 0, semA)

            wait(i0 + 1, 1, semB)
            acc_row(i0 + 1, 1)
            return carry

        lax.fori_loop(0, ROWS_PER_W // 2, pair_body, 0)
        pltpu.sync_copy(out_v, out_hbm.at[pl.ds(base, ROWS_PER_W)])

    return k(tab_i32, idx4)


def _mlp_tc(pooled, inp, t0p, W1t, b1, W2t, b2, W3t, b3):
    BB = 2048

    def mlp_kernel(p_ref, idx_ref, t0_ref, w1_ref, b1_ref, w2_ref, b2_ref,
                   w3_ref, b3_ref, o_ref):
        cnt = jnp.sum((idx_ref[...] == 0).astype(jnp.float32), axis=1,
                      keepdims=True)
        bow = (p_ref[...] - cnt * t0_ref[...]) * (1.0 / SEQ)
        h = jnp.maximum(
            jnp.dot(bow, w1_ref[...], preferred_element_type=jnp.float32)
            + b1_ref[...], 0.0)
        h = jnp.maximum(
            jnp.dot(h, w2_ref[...], preferred_element_type=jnp.float32)
            + b2_ref[...], 0.0)
        o_ref[...] = (
            jnp.dot(h, w3_ref[...], preferred_element_type=jnp.float32)
            + b3_ref[...])

    return pl.pallas_call(
        mlp_kernel,
        grid=(BATCH // BB,),
        in_specs=[
            pl.BlockSpec((BB, HIDDEN), lambda i: (i, 0)),
            pl.BlockSpec((BB, SEQ), lambda i: (i, 0)),
            pl.BlockSpec((1, HIDDEN), lambda i: (0, 0)),
            pl.BlockSpec((HIDDEN, HIDDEN), lambda i: (0, 0)),
            pl.BlockSpec((1, HIDDEN), lambda i: (0, 0)),
            pl.BlockSpec((HIDDEN, HIDDEN), lambda i: (0, 0)),
            pl.BlockSpec((1, HIDDEN), lambda i: (0, 0)),
            pl.BlockSpec((HIDDEN, LABELS), lambda i: (0, 0)),
            pl.BlockSpec((1, LABELS), lambda i: (0, 0)),
        ],
        out_specs=pl.BlockSpec((BB, LABELS), lambda i: (i, 0)),
        out_shape=jax.ShapeDtypeStruct((BATCH, LABELS), jnp.float32),
    )(pooled, inp, t0p, W1t, b1, W2t, b2, W3t, b3)


def kernel(input, table, W1, b1, W2, b2, W3, b3):
    inp = input.astype(jnp.int32)
    idx4 = inp.reshape(NW, ROWS_PER_W, 2, HALF)
    tab_i32 = _pack_table_tc(table).reshape(VOCAB, WORDS)
    pooled = _pooled_sum_sc(tab_i32, idx4)
    w0 = tab_i32[0:1]
    t0p = jnp.concatenate([
        lax.bitcast_convert_type(w0 << 16, jnp.float32),
        lax.bitcast_convert_type(w0 & _MASK, jnp.float32),
    ], axis=1)
    return _mlp_tc(
        pooled, inp, t0p,
        W1.T, b1.reshape(1, HIDDEN),
        W2.T, b2.reshape(1, HIDDEN),
        W3.T, b3.reshape(1, LABELS),
    )
